# baseline (speedup 1.0000x reference)
import jax
import jax.numpy as jnp
from jax.experimental import pallas as pl
from jax.experimental.pallas import tpu as pltpu

F_FRAMES = 4
KF = 16
_SIGNS = jnp.array(
    [[1., 1., 1.], [-1., -1., 1.], [-1., 1., -1.], [1., -1., -1.]],
    dtype=jnp.float32)


def _pdist2(a, b):
    return jnp.sum((a[:, :, None, :] - b[:, None, :, :]) ** 2, axis=-1)


def _gather(src, idx):
    return jax.vmap(lambda s, i: s[i])(src, idx)


def _bn(x, g, b):
    m = x.mean(axis=(0, 1), keepdims=True)
    v = x.var(axis=(0, 1), keepdims=True)
    return (x - m) / jnp.sqrt(v + 1e-5) * g + b


def _frames(pq, pa):
    d2 = _pdist2(pq, pa)
    _, idx = jax.lax.top_k(-d2, KF)
    nb = _gather(pa, idx)
    c = nb - nb.mean(axis=2, keepdims=True)
    cov = jnp.einsum('bqki,bqkj->bqij', c, c) / KF
    _, V = jnp.linalg.eigh(cov)
    Vt = jnp.swapaxes(V, -1, -2)
    R = _SIGNS[None, None, :, :, None] * Vt[:, :, None, :, :]
    return R.reshape(R.shape[0], R.shape[1], F_FRAMES * 3, 3)


def _rsconv(p_src, p_dst, R_dst, h, k, Wm1, bm1, Wm2, bm2, Wr, g, b):
    d2 = _pdist2(p_dst, p_src)
    _, idx = jax.lax.top_k(-d2, k)
    nb_p = _gather(p_src, idx)
    nb_h = _gather(h, idx)
    rel = nb_p - p_dst[:, :, None, :]
    orel = jnp.einsum('bmfj,bmkj->bmkf', R_dst, rel)
    dist = jnp.sqrt(jnp.sum(rel ** 2, axis=-1, keepdims=True) + 1e-9)
    geo = jnp.concatenate([dist, orel], axis=-1)
    w = jax.nn.relu(geo @ Wm1 + bm1) @ Wm2 + bm2
    agg = jnp.max(w * nb_h, axis=2)
    return jax.nn.relu(_bn(agg @ Wr, g, b))


def _head_body(h_ref, wc1_ref, gc1_ref, bec1_ref, wc2_ref, gc2_ref,
               bec2_ref, wc3_ref, bc3_ref, out_ref):
    def bn2(x, g, bb):
        m = jnp.mean(x, axis=0, keepdims=True)
        v = jnp.mean((x - m) ** 2, axis=0, keepdims=True)
        return (x - m) / jnp.sqrt(v + 1e-5) * g + bb

    x = h_ref[...] @ wc1_ref[...]
    x = bn2(x, gc1_ref[...], bec1_ref[...])
    x = x @ wc2_ref[...]
    x = bn2(x, gc2_ref[...], bec2_ref[...])
    out_ref[...] = x @ wc3_ref[...] + bc3_ref[...][None, :]


def _head(h, P):
    B = h.shape[0]
    return pl.pallas_call(
        _head_body,
        out_shape=jax.ShapeDtypeStruct((B, 40), jnp.float32),
    )(h, P['Wc1'], P['gc1'], P['bec1'], P['Wc2'], P['gc2'], P['bec2'],
      P['Wc3'], P['bc3'])


def kernel(p, params):
    P = params
    p1 = p[:, ::2]
    p2 = p[:, ::8]
    p3 = p.mean(axis=1, keepdims=True)
    R1 = _frames(p1, p)
    R2 = _frames(p2, p)
    R3 = _frames(p3, p)
    N = p.shape[1]
    pc = jnp.tile(p3, (1, N, 1))
    Rc = jnp.tile(R3, (1, N, 1, 1))
    pg = jnp.einsum('bnfj,bnj->bnf', Rc, p - pc)
    h = jax.nn.relu(_bn(pg @ P['Wxr'] + P['bxr'], P['gxr'], P['bexr']))
    h = _rsconv(p, p1, R1, h, 48, P['W1m1'], P['b1m1'], P['W1m2'],
                P['b1m2'], P['W1r'], P['g1'], P['be1'])
    h = _rsconv(p1, p2, R2, h, 64, P['W2m1'], P['b2m1'], P['W2m2'],
                P['b2m2'], P['W2r'], P['g2'], P['be2'])
    h = _rsconv(p2, p3, R3, h, 128, P['W3m1'], P['b3m1'], P['W3m2'],
                P['b3m2'], P['W3r'], P['g3'], P['be3'])
    return _head(h[:, 0, :], P)
